# baseline (speedup 1.0000x reference)
import functools

import jax
import jax.numpy as jnp
from jax import lax
from jax.experimental import pallas as pl
from jax.experimental.pallas import tpu as pltpu
from jax.experimental.pallas import tpu_sc as plsc

T, NS, NW, NQ, D = 32, 320, 16, 240, 512
NC, NSUB = 2, 16
CHUNK = 64
NCHUNK = NS // CHUNK
NG = CHUNK // 16
DV = D // 16


@functools.partial(
    pl.kernel,
    mesh=plsc.VectorSubcoreMesh(core_axis_name="c", subcore_axis_name="s"),
    out_type=jax.ShapeDtypeStruct((T * NW, D), jnp.float32),
    scratch_types=[
        pltpu.VMEM((NS + 16,), jnp.int32),
        pltpu.VMEM((2, CHUNK, D), jnp.float32),
        pltpu.VMEM((NW, D), jnp.float32),
        pltpu.SemaphoreType.DMA,
        pltpu.SemaphoreType.DMA,
    ],
)
def _sc_psums(sup_hbm, lab_hbm, out_hbm, lab_v, chunk_v, acc_v, sem0, sem1):
    c = lax.axis_index("c")
    s = lax.axis_index("s")
    t = c * NSUB + s
    sems = (sem0, sem1)

    zero = jnp.zeros((16,), jnp.float32)
    for w in range(NW):
        for d in range(DV):
            acc_v[w, pl.ds(d * 16, 16)] = zero

    pltpu.sync_copy(lab_hbm.at[t], lab_v)

    def start(k):
        return pltpu.async_copy(
            sup_hbm.at[pl.ds(t * NS + k * CHUNK, CHUNK)],
            chunk_v.at[k % 2], sems[k % 2])

    copies = {0: start(0)}
    for k in range(NCHUNK):
        if k + 1 < NCHUNK:
            copies[k + 1] = start(k + 1)
        copies[k].wait()
        b = k % 2

        def group_body(g, carry, k=k, b=b):
            lab16 = lab_v[pl.ds(k * CHUNK + g * 16, 16)]
            for j in range(16):
                l = lab16[j]
                row = g * 16 + j
                for d in range(DV):
                    plsc.addupdate(acc_v.at[l, pl.ds(d * 16, 16)],
                                   chunk_v[b, row, pl.ds(d * 16, 16)])
            return carry

        lax.fori_loop(0, NG, group_body, 0)

    pltpu.sync_copy(acc_v, out_hbm.at[pl.ds(t * NW, NW)])


def _tc_body(lab_ref, ps_ref, qry_ref, out_ref):
    lab = lab_ref[0]
    ps = ps_ref[0]
    q = qry_ref[0]

    wids = lax.broadcasted_iota(jnp.int32, (NW, NS), 0)
    oh = jnp.where(wids == lab, 1.0, 0.0).astype(jnp.float32)
    cnt = jnp.sum(oh, axis=1, keepdims=True)
    protos = ps / cnt

    wn = jnp.sum(protos * protos, axis=1, keepdims=True)
    scores = lax.dot_general(
        protos, q, (((1,), (1,)), ((), ())),
        preferred_element_type=jnp.float32,
        precision=lax.Precision.HIGHEST)
    d2 = wn - 2.0 * scores

    idx = lax.broadcasted_iota(jnp.int32, (NW, NQ), 0)
    m = jnp.min(d2, axis=0, keepdims=True)
    pred = jnp.min(jnp.where(d2 == m, idx, NW), axis=0, keepdims=True)
    out_ref[0] = pred.astype(jnp.int32)


@jax.jit
def kernel(support_features, support_labels, query_features):
    labels_pad = jnp.pad(support_labels, ((0, 0), (0, 16)))
    psums = _sc_psums(support_features.reshape(T * NS, D),
                      labels_pad).reshape(T, NW, D)
    labels3 = support_labels.reshape(T, 1, NS)
    out = pl.pallas_call(
        _tc_body,
        grid=(T,),
        in_specs=[
            pl.BlockSpec((1, 1, NS), lambda t: (t, 0, 0)),
            pl.BlockSpec((1, NW, D), lambda t: (t, 0, 0)),
            pl.BlockSpec((1, NQ, D), lambda t: (t, 0, 0)),
        ],
        out_specs=pl.BlockSpec((1, 1, NQ), lambda t: (t, 0, 0)),
        out_shape=jax.ShapeDtypeStruct((T, 1, NQ), jnp.int32),
    )(labels3, psums, query_features)
    return out.reshape(T, NQ)


# speedup vs baseline: 3.6420x; 3.6420x over previous
import jax
import jax.numpy as jnp
from jax import lax
from jax.experimental import pallas as pl

T, NS, NW, NQ, D = 32, 320, 16, 240, 512
TB = 4


def _body(lab_ref, sup_ref, qry_ref, out_ref):
    for b in range(TB):
        lab = lab_ref[b]
        sup = sup_ref[b]
        q = qry_ref[b]

        wids = lax.broadcasted_iota(jnp.int32, (NW, NS), 0)
        oh = jnp.where(wids == lab, 1.0, 0.0).astype(jnp.float32)
        cnt = jnp.sum(oh, axis=1, keepdims=True)
        psum = lax.dot_general(
            oh, sup, (((1,), (0,)), ((), ())),
            preferred_element_type=jnp.float32,
            precision=lax.Precision.HIGHEST)
        protos = psum / cnt

        wn = jnp.sum(protos * protos, axis=1, keepdims=True)
        scores = lax.dot_general(
            protos, q, (((1,), (1,)), ((), ())),
            preferred_element_type=jnp.float32,
            precision=lax.Precision.HIGHEST)
        d2 = wn - 2.0 * scores

        idx = lax.broadcasted_iota(jnp.int32, (NW, NQ), 0)
        m = jnp.min(d2, axis=0, keepdims=True)
        pred = jnp.min(jnp.where(d2 == m, idx, NW), axis=0, keepdims=True)
        out_ref[b] = pred.astype(jnp.int32)


@jax.jit
def kernel(support_features, support_labels, query_features):
    labels3 = support_labels.reshape(T, 1, NS)
    out = pl.pallas_call(
        _body,
        grid=(T // TB,),
        in_specs=[
            pl.BlockSpec((TB, 1, NS), lambda t: (t, 0, 0)),
            pl.BlockSpec((TB, NS, D), lambda t: (t, 0, 0)),
            pl.BlockSpec((TB, NQ, D), lambda t: (t, 0, 0)),
        ],
        out_specs=pl.BlockSpec((TB, 1, NQ), lambda t: (t, 0, 0)),
        out_shape=jax.ShapeDtypeStruct((T, 1, NQ), jnp.int32),
    )(labels3, support_features, query_features)
    return out.reshape(T, NQ)
